# baseline (speedup 1.0000x reference)
import jax
import jax.numpy as jnp
from jax.experimental import pallas as pl
from jax.experimental.pallas import tpu as pltpu

_B, _C, _H, _W = 32, 96, 224, 224
_CCHUNK = 16
_NC = _C // _CCHUNK
_TOPKS = (3, 4, 5, 6, 7, 10, 15, 20)


def _fused_kernel(in_ref, w1_ref, w2_ref, x_sun_ref, x_son_ref, copy_ref,
                  acc_ref):
    b = pl.program_id(0)
    c = pl.program_id(1)
    x = in_ref[...]
    copy_ref[...] = x
    rows = jnp.sum(x, axis=2)[0]

    for ci in range(_NC):
        lo = ci * _CCHUNK
        if lo >= 40:
            break
        n = min(_CCHUNK, 40 - lo)

        @pl.when(c == ci)
        def _(lo=lo, n=n):
            acc_ref[b, lo:lo + n, :] = rows[0:n, :]

    @pl.when(jnp.logical_and(b == _B - 1, c == _NC - 1))
    def _():
        p = acc_ref[:, :33, :]
        sums = jnp.sum(p, axis=2)
        x_sun = sums * (1.0 / (_H * _W))
        x_sun_ref[...] = x_sun

        xt = jnp.tanh(x_sun)
        xg = jax.lax.dot_general(
            xt, w1_ref[...], (((1,), (1,)), ((), ())),
            preferred_element_type=jnp.float32)
        xg = jnp.maximum(xg, 0.0)

        votes = xg[:, None, :] * w2_ref[...][None, :, :]
        x_son_ref[8] = jnp.sum(votes, axis=2)

        nin = votes.shape[2]
        idx = jax.lax.broadcasted_iota(jnp.int32, votes.shape, 2)
        v = votes
        acc = jnp.zeros(votes.shape[:2], jnp.float32)
        kslot = {k: i for i, k in enumerate(_TOPKS)}
        for i in range(1, max(_TOPKS) + 1):
            m = jnp.max(v, axis=2)
            acc = acc + m
            if i in kslot:
                x_son_ref[kslot[i]] = acc
            eq = v == m[:, :, None]
            first = jnp.min(jnp.where(eq, idx, nin), axis=2)
            v = jnp.where(idx == first[:, :, None], -jnp.inf, v)


def kernel(maps, W1, W2):
    x_sun, x_son, maps_copy = pl.pallas_call(
        _fused_kernel,
        grid=(_B, _NC),
        in_specs=[pl.BlockSpec((1, _CCHUNK, _H, _W),
                               lambda b, c: (b, c, 0, 0)),
                  pl.BlockSpec(W1.shape, lambda b, c: (0, 0)),
                  pl.BlockSpec(W2.shape, lambda b, c: (0, 0))],
        out_specs=[pl.BlockSpec((_B, 33), lambda b, c: (0, 0)),
                   pl.BlockSpec((9, _B, 10), lambda b, c: (0, 0, 0)),
                   pl.BlockSpec((1, _CCHUNK, _H, _W),
                                lambda b, c: (b, c, 0, 0))],
        out_shape=[jax.ShapeDtypeStruct((_B, 33), jnp.float32),
                   jax.ShapeDtypeStruct((9, _B, 10), jnp.float32),
                   jax.ShapeDtypeStruct((_B, _C, _H, _W), jnp.float32)],
        scratch_shapes=[pltpu.VMEM((_B, 40, _W), jnp.float32)],
        compiler_params=pltpu.CompilerParams(
            dimension_semantics=("arbitrary", "arbitrary")),
    )(maps, W1, W2)

    return (x_sun, x_son, maps_copy)


# speedup vs baseline: 1.0257x; 1.0257x over previous
import jax
import jax.numpy as jnp
from jax.experimental import pallas as pl
from jax.experimental.pallas import tpu as pltpu

_B, _C, _H, _W = 32, 96, 224, 224
_CCHUNK = 48
_NC = _C // _CCHUNK
_TOPKS = (3, 4, 5, 6, 7, 10, 15, 20)


def _fused_kernel(in_ref, w1_ref, w2_ref, x_sun_ref, x_son_ref, copy_ref,
                  acc_ref):
    c = pl.program_id(0)
    b = pl.program_id(1)
    x = in_ref[...]
    copy_ref[...] = x
    rows = jnp.sum(x, axis=2)[0]

    for ci in range(_NC):
        lo = ci * _CCHUNK
        if lo >= 40:
            break
        n = min(_CCHUNK, 40 - lo)

        @pl.when(c == ci)
        def _(lo=lo, n=n):
            acc_ref[b, lo:lo + n, :] = rows[0:n, :]

    @pl.when(jnp.logical_and(c == 0, b == _B - 1))
    def _():
        p = acc_ref[:, :33, :]
        sums = jnp.sum(p, axis=2)
        x_sun = sums * (1.0 / (_H * _W))
        x_sun_ref[...] = x_sun

        xt = jnp.tanh(x_sun)
        xg = jax.lax.dot_general(
            xt, w1_ref[...], (((1,), (1,)), ((), ())),
            preferred_element_type=jnp.float32)
        xg = jnp.maximum(xg, 0.0)

        votes = xg[:, None, :] * w2_ref[...][None, :, :]
        x_son_ref[8] = jnp.sum(votes, axis=2)

        nin = votes.shape[2]
        idx = jax.lax.broadcasted_iota(jnp.int32, votes.shape, 2)
        v = votes
        acc = jnp.zeros(votes.shape[:2], jnp.float32)
        kslot = {k: i for i, k in enumerate(_TOPKS)}
        for i in range(1, max(_TOPKS) + 1):
            m = jnp.max(v, axis=2)
            acc = acc + m
            if i in kslot:
                x_son_ref[kslot[i]] = acc
            eq = v == m[:, :, None]
            first = jnp.min(jnp.where(eq, idx, nin), axis=2)
            v = jnp.where(idx == first[:, :, None], -jnp.inf, v)


def kernel(maps, W1, W2):
    x_sun, x_son, maps_copy = pl.pallas_call(
        _fused_kernel,
        grid=(_NC, _B),
        in_specs=[pl.BlockSpec((1, _CCHUNK, _H, _W),
                               lambda c, b: (b, c, 0, 0)),
                  pl.BlockSpec(W1.shape, lambda c, b: (0, 0)),
                  pl.BlockSpec(W2.shape, lambda c, b: (0, 0))],
        out_specs=[pl.BlockSpec((_B, 33), lambda c, b: (0, 0)),
                   pl.BlockSpec((9, _B, 10), lambda c, b: (0, 0, 0)),
                   pl.BlockSpec((1, _CCHUNK, _H, _W),
                                lambda c, b: (b, c, 0, 0))],
        out_shape=[jax.ShapeDtypeStruct((_B, 33), jnp.float32),
                   jax.ShapeDtypeStruct((9, _B, 10), jnp.float32),
                   jax.ShapeDtypeStruct((_B, _C, _H, _W), jnp.float32)],
        scratch_shapes=[pltpu.VMEM((_B, 40, _W), jnp.float32)],
        compiler_params=pltpu.CompilerParams(
            dimension_semantics=("arbitrary", "arbitrary")),
    )(maps, W1, W2)

    return (x_sun, x_son, maps_copy)
